# baseline (speedup 1.0000x reference)
import jax
import jax.numpy as jnp
from jax import lax
from jax.experimental import pallas as pl
from jax.experimental.pallas import tpu as pltpu
from jax.experimental.pallas import tpu_sc as plsc

_N, _C, _H, _W = 4, 96, 224, 224
_P = _H * _W
_B = _N * _P
_NW = 32
_PW = _B // _NW
_CH = 64
_NCHUNK = _PW // _CH
_L = 16

_SLOT_KEYS = ("i00", "i01", "i10", "i11",
              "w00", "w01", "w10", "w11",
              "r00", "r01", "r10", "r11", "out", "sem", "osem")


def _slot_scratch():
    return [
        pltpu.VMEM((_CH,), jnp.int32),
        pltpu.VMEM((_CH,), jnp.int32),
        pltpu.VMEM((_CH,), jnp.int32),
        pltpu.VMEM((_CH,), jnp.int32),
        pltpu.VMEM((_CH,), jnp.float32),
        pltpu.VMEM((_CH,), jnp.float32),
        pltpu.VMEM((_CH,), jnp.float32),
        pltpu.VMEM((_CH,), jnp.float32),
        pltpu.VMEM((_CH, _C), jnp.float32),
        pltpu.VMEM((_CH, _C), jnp.float32),
        pltpu.VMEM((_CH, _C), jnp.float32),
        pltpu.VMEM((_CH, _C), jnp.float32),
        pltpu.VMEM((_CH, _C), jnp.float32),
        pltpu.SemaphoreType.DMA,
        pltpu.SemaphoreType.DMA,
    ]


def _warp_body(table, g_hbm, out_hbm, gall, *scratch):
    nslot = len(_SLOT_KEYS)
    slots = [dict(zip(_SLOT_KEYS, scratch[k * nslot:(k + 1) * nslot]))
             for k in range(2)]

    wid = lax.axis_index("c") * 16 + lax.axis_index("s")
    base = wid * _PW
    nb = (wid // 8) * _P
    iota = lax.iota(jnp.int32, _L)
    iota2 = iota * 2

    pltpu.sync_copy(g_hbm.at[pl.ds(2 * base, 2 * _PW)], gall)

    def prep_fire(ch, S):
        goff = 2 * ch * _CH
        for i in range(_CH // _L):
            gxv = plsc.load_gather(gall, [iota2 + (goff + 2 * _L * i)])
            gyv = plsc.load_gather(gall, [iota2 + (goff + 2 * _L * i + 1)])
            ix = (gxv + 1.0) * (_W * 0.5) - 0.5
            iy = (gyv + 1.0) * (_H * 0.5) - 0.5
            ix = jnp.minimum(jnp.maximum(ix, 0.0), _W - 1.0)
            iy = jnp.minimum(jnp.maximum(iy, 0.0), _H - 1.0)
            ix0 = ix.astype(jnp.int32)
            iy0 = iy.astype(jnp.int32)
            wx1 = ix - ix0.astype(jnp.float32)
            wy1 = iy - iy0.astype(jnp.float32)
            wx0 = 1.0 - wx1
            wy0 = 1.0 - wy1
            ix1 = jnp.minimum(ix0 + 1, _W - 1)
            iy1 = jnp.minimum(iy0 + 1, _H - 1)
            row0 = nb + iy0 * _W
            row1 = nb + iy1 * _W
            s = i * _L
            S["i00"][pl.ds(s, _L)] = row0 + ix0
            S["i01"][pl.ds(s, _L)] = row0 + ix1
            S["i10"][pl.ds(s, _L)] = row1 + ix0
            S["i11"][pl.ds(s, _L)] = row1 + ix1
            S["w00"][pl.ds(s, _L)] = wy0 * wx0
            S["w01"][pl.ds(s, _L)] = wy0 * wx1
            S["w10"][pl.ds(s, _L)] = wy1 * wx0
            S["w11"][pl.ds(s, _L)] = wy1 * wx1
        for ib, rb in (("i00", "r00"), ("i01", "r01"),
                       ("i10", "r10"), ("i11", "r11")):
            pltpu.async_copy(table.at[S[ib]], S[rb], S["sem"])

    def wait_gathers(S):
        for ib, rb in (("i00", "r00"), ("i01", "r01"),
                       ("i10", "r10"), ("i11", "r11")):
            pltpu.make_async_copy(table.at[S[ib]], S[rb], S["sem"]).wait()

    def compute_out(ch, S):
        @pl.when(ch >= 2)
        def _():
            pltpu.make_async_copy(
                S["out"], out_hbm.at[pl.ds(base + (ch - 2) * _CH, _CH)],
                S["osem"]).wait()

        def pix(p, _):
            for u in range(2):
                pp = p * 2 + u
                pv = jnp.broadcast_to(pp, (_L,))
                v00 = plsc.load_gather(S["w00"], [pv])
                v01 = plsc.load_gather(S["w01"], [pv])
                v10 = plsc.load_gather(S["w10"], [pv])
                v11 = plsc.load_gather(S["w11"], [pv])
                for j in range(_C // _L):
                    cs = pl.ds(j * _L, _L)
                    a = S["r00"][pp, cs] * v00
                    a = a + S["r01"][pp, cs] * v01
                    a = a + S["r10"][pp, cs] * v10
                    a = a + S["r11"][pp, cs] * v11
                    S["out"][pp, cs] = a
            return 0

        lax.fori_loop(0, _CH // 2, pix, 0)
        pltpu.async_copy(S["out"], out_hbm.at[pl.ds(base + ch * _CH, _CH)],
                         S["osem"])

    prep_fire(0, slots[0])

    def step(j, _):
        prep_fire(2 * j + 1, slots[1])
        wait_gathers(slots[0])
        compute_out(2 * j, slots[0])

        @pl.when(j < _NCHUNK // 2 - 1)
        def _():
            prep_fire(2 * j + 2, slots[0])

        wait_gathers(slots[1])
        compute_out(2 * j + 1, slots[1])
        return 0

    lax.fori_loop(0, _NCHUNK // 2, step, 0)

    for ch, S in ((_NCHUNK - 2, slots[0]), (_NCHUNK - 1, slots[1])):
        pltpu.make_async_copy(
            S["out"], out_hbm.at[pl.ds(base + ch * _CH, _CH)],
            S["osem"]).wait()


def _make_warp():
    mesh = plsc.VectorSubcoreMesh(core_axis_name="c", subcore_axis_name="s")
    return pl.kernel(
        _warp_body,
        mesh=mesh,
        compiler_params=pltpu.CompilerParams(
            needs_layout_passes=False, use_tc_tiling_on_sc=False),
        out_type=jax.ShapeDtypeStruct((_B, _C), jnp.float32),
        scratch_types=[pltpu.VMEM((2 * _PW,), jnp.float32)]
        + _slot_scratch() + _slot_scratch(),
    )


_TSP = 3584
_TGRID = _P // _TSP


def _tr_body(x_ref, o_ref):
    o_ref[...] = jnp.transpose(x_ref[0], (1, 0))


def _make_table_transpose():
    return pl.pallas_call(
        _tr_body,
        grid=(_N, _TGRID),
        in_specs=[pl.BlockSpec((1, _C, _TSP), lambda n, s: (n, 0, s))],
        out_specs=pl.BlockSpec((_TSP, _C), lambda n, s: (n * _TGRID + s, 0)),
        out_shape=jax.ShapeDtypeStruct((_B, _C), jnp.float32),
    )


def _tr_out_body(x_ref, o_ref):
    o_ref[...] = jnp.transpose(x_ref[...], (1, 0))[None]


def _make_out_transpose():
    return pl.pallas_call(
        _tr_out_body,
        grid=(_N, _TGRID),
        in_specs=[pl.BlockSpec((_TSP, _C), lambda n, s: (n * _TGRID + s, 0))],
        out_specs=pl.BlockSpec((1, _C, _TSP), lambda n, s: (n, 0, s)),
        out_shape=jax.ShapeDtypeStruct((_N, _C, _P), jnp.float32),
    )


@jax.jit
def kernel(inputs, grid):
    n, c, h, w = inputs.shape
    table = _make_table_transpose()(inputs.reshape(n, c, h * w))
    out = _make_warp()(table, grid.reshape(n * h * w * 2))
    return _make_out_transpose()(out).reshape(n, c, h, w)


# speedup vs baseline: 1.4227x; 1.4227x over previous
import jax
import jax.numpy as jnp
from jax import lax
from jax.experimental import pallas as pl
from jax.experimental.pallas import tpu as pltpu
from jax.experimental.pallas import tpu_sc as plsc

_N, _C, _H, _W = 4, 96, 224, 224
_P = _H * _W
_B = _N * _P
_NW = 32
_PW = _B // _NW
_CH = 64
_NCHUNK = _PW // _CH
_L = 16

_SLOT_KEYS = ("i00", "i01", "i10", "i11", "wx", "wy",
              "r00", "r01", "r10", "r11", "out", "sem", "osem")


def _slot_scratch():
    return [
        pltpu.VMEM((_CH,), jnp.int32),
        pltpu.VMEM((_CH,), jnp.int32),
        pltpu.VMEM((_CH,), jnp.int32),
        pltpu.VMEM((_CH,), jnp.int32),
        pltpu.VMEM((_CH,), jnp.float32),
        pltpu.VMEM((_CH,), jnp.float32),
        pltpu.VMEM((_CH, _C), jnp.float32),
        pltpu.VMEM((_CH, _C), jnp.float32),
        pltpu.VMEM((_CH, _C), jnp.float32),
        pltpu.VMEM((_CH, _C), jnp.float32),
        pltpu.VMEM((_CH, _C), jnp.float32),
        pltpu.SemaphoreType.DMA,
        pltpu.SemaphoreType.DMA,
    ]


def _warp_body(table, g_hbm, out_hbm, gall, *scratch):
    nslot = len(_SLOT_KEYS)
    slots = [dict(zip(_SLOT_KEYS, scratch[k * nslot:(k + 1) * nslot]))
             for k in range(3)]

    wid = lax.axis_index("c") * 16 + lax.axis_index("s")
    base = wid * _PW
    nb = (wid // 8) * _P
    iota = lax.iota(jnp.int32, _L)
    iota2 = iota * 2

    pltpu.sync_copy(g_hbm.at[pl.ds(2 * base, 2 * _PW)], gall)

    def prep_fire(ch, S):
        goff = 2 * ch * _CH
        for i in range(_CH // _L):
            gxv = plsc.load_gather(gall, [iota2 + (goff + 2 * _L * i)])
            gyv = plsc.load_gather(gall, [iota2 + (goff + 2 * _L * i + 1)])
            ix = (gxv + 1.0) * (_W * 0.5) - 0.5
            iy = (gyv + 1.0) * (_H * 0.5) - 0.5
            ix = jnp.minimum(jnp.maximum(ix, 0.0), _W - 1.0)
            iy = jnp.minimum(jnp.maximum(iy, 0.0), _H - 1.0)
            ix0 = ix.astype(jnp.int32)
            iy0 = iy.astype(jnp.int32)
            wx1 = ix - ix0.astype(jnp.float32)
            wy1 = iy - iy0.astype(jnp.float32)
            ix1 = jnp.minimum(ix0 + 1, _W - 1)
            iy1 = jnp.minimum(iy0 + 1, _H - 1)
            row0 = nb + iy0 * _W
            row1 = nb + iy1 * _W
            s = i * _L
            S["i00"][pl.ds(s, _L)] = row0 + ix0
            S["i01"][pl.ds(s, _L)] = row0 + ix1
            S["i10"][pl.ds(s, _L)] = row1 + ix0
            S["i11"][pl.ds(s, _L)] = row1 + ix1
            S["wx"][pl.ds(s, _L)] = wx1
            S["wy"][pl.ds(s, _L)] = wy1
        for ib, rb in (("i00", "r00"), ("i01", "r01"),
                       ("i10", "r10"), ("i11", "r11")):
            pltpu.async_copy(table.at[S[ib]], S[rb], S["sem"])

    def wait_gathers(S):
        for ib, rb in (("i00", "r00"), ("i01", "r01"),
                       ("i10", "r10"), ("i11", "r11")):
            pltpu.make_async_copy(table.at[S[ib]], S[rb], S["sem"]).wait()

    def compute_out(ch, S):
        @pl.when(ch >= 3)
        def _():
            pltpu.make_async_copy(
                S["out"], out_hbm.at[pl.ds(base + (ch - 3) * _CH, _CH)],
                S["osem"]).wait()

        def pix(p, _):
            for u in range(4):
                pp = p * 4 + u
                pv = jnp.broadcast_to(pp, (_L,))
                vx = plsc.load_gather(S["wx"], [pv])
                vy = plsc.load_gather(S["wy"], [pv])
                for j in range(_C // _L):
                    cs = pl.ds(j * _L, _L)
                    l00 = S["r00"][pp, cs]
                    l01 = S["r01"][pp, cs]
                    l10 = S["r10"][pp, cs]
                    l11 = S["r11"][pp, cs]
                    top = l00 + vx * (l01 - l00)
                    bot = l10 + vx * (l11 - l10)
                    S["out"][pp, cs] = top + vy * (bot - top)
            return 0

        lax.fori_loop(0, _CH // 4, pix, 0)
        pltpu.async_copy(S["out"], out_hbm.at[pl.ds(base + ch * _CH, _CH)],
                         S["osem"])

    prep_fire(0, slots[0])
    prep_fire(1, slots[1])

    def step(j, _):
        c = 3 * j
        prep_fire(c + 2, slots[2])
        wait_gathers(slots[0])
        compute_out(c, slots[0])
        prep_fire(c + 3, slots[0])
        wait_gathers(slots[1])
        compute_out(c + 1, slots[1])
        prep_fire(c + 4, slots[1])
        wait_gathers(slots[2])
        compute_out(c + 2, slots[2])
        return 0

    lax.fori_loop(0, (_NCHUNK - 2) // 3, step, 0)

    wait_gathers(slots[0])
    compute_out(_NCHUNK - 2, slots[0])
    wait_gathers(slots[1])
    compute_out(_NCHUNK - 1, slots[1])

    for ch, S in ((_NCHUNK - 3, slots[2]), (_NCHUNK - 2, slots[0]),
                  (_NCHUNK - 1, slots[1])):
        pltpu.make_async_copy(
            S["out"], out_hbm.at[pl.ds(base + ch * _CH, _CH)],
            S["osem"]).wait()


def _make_warp():
    mesh = plsc.VectorSubcoreMesh(core_axis_name="c", subcore_axis_name="s")
    return pl.kernel(
        _warp_body,
        mesh=mesh,
        compiler_params=pltpu.CompilerParams(
            needs_layout_passes=False, use_tc_tiling_on_sc=False),
        out_type=jax.ShapeDtypeStruct((_B, _C), jnp.float32),
        scratch_types=[pltpu.VMEM((2 * _PW,), jnp.float32)]
        + _slot_scratch() + _slot_scratch() + _slot_scratch(),
    )


_TSP = 3584
_TGRID = _P // _TSP


def _tr_body(x_ref, o_ref):
    o_ref[...] = jnp.transpose(x_ref[0], (1, 0))


def _make_table_transpose():
    return pl.pallas_call(
        _tr_body,
        grid=(_N, _TGRID),
        in_specs=[pl.BlockSpec((1, _C, _TSP), lambda n, s: (n, 0, s))],
        out_specs=pl.BlockSpec((_TSP, _C), lambda n, s: (n * _TGRID + s, 0)),
        out_shape=jax.ShapeDtypeStruct((_B, _C), jnp.float32),
    )


@jax.jit
def kernel(inputs, grid):
    n, c, h, w = inputs.shape
    table = _make_table_transpose()(inputs.reshape(n, c, h * w))
    out = _make_warp()(table, grid.reshape(n * h * w * 2))
    return jnp.transpose(out.reshape(n, h, w, c), (0, 3, 1, 2))
